# Initial kernel scaffold; baseline (speedup 1.0000x reference)
import jax
import jax.numpy as jnp
from jax import lax
from jax.experimental import pallas as pl
from jax.experimental.pallas import tpu as pltpu
from jax.experimental.pallas import tpu_sc as plsc

N_EMB = 1000000
EMB_DIM = 32
EMB_SCALE = 10.0

B_TOTAL = 16384 * 50
NUM_WORKERS = 32
B_PER_W = B_TOTAL // NUM_WORKERS
CHUNK = 1280
NCHUNK = B_PER_W // CHUNK


def _sc_kernel(x_hbm, table_hbm, out_hbm, idx_v, rows_v, sem):
    cid = lax.axis_index("c")
    sid = lax.axis_index("s")
    wid = sid * 2 + cid
    base = wid * B_PER_W

    @pl.loop(0, NCHUNK)
    def _chunk(g):
        off = base + g * CHUNK
        pltpu.sync_copy(x_hbm.at[pl.ds(off, CHUNK)], idx_v)
        pltpu.async_copy(table_hbm.at[idx_v], rows_v, sem).wait()

        @pl.loop(0, CHUNK, unroll=4)
        def _scale(i):
            rows_v[i, pl.ds(0, 16)] = rows_v[i, pl.ds(0, 16)] * EMB_SCALE
            rows_v[i, pl.ds(16, 16)] = rows_v[i, pl.ds(16, 16)] * EMB_SCALE

        pltpu.sync_copy(rows_v, out_hbm.at[pl.ds(off, CHUNK)])


@jax.jit
def _scaled_embedding(x_flat, table):
    mesh = plsc.VectorSubcoreMesh(core_axis_name="c", subcore_axis_name="s")
    return pl.kernel(
        _sc_kernel,
        out_type=jax.ShapeDtypeStruct((B_TOTAL, EMB_DIM), jnp.float32),
        mesh=mesh,
        scratch_types=[
            pltpu.VMEM((CHUNK,), jnp.int32),
            pltpu.VMEM((CHUNK, EMB_DIM), jnp.float32),
            pltpu.SemaphoreType.DMA,
        ],
    )(x_flat, table)


def kernel(x, table):
    out = _scaled_embedding(x.reshape(-1).astype(jnp.int32), table)
    return out.reshape(x.shape + (EMB_DIM,))


# speedup vs baseline: 1.0201x; 1.0201x over previous
import jax
import jax.numpy as jnp
from jax import lax
from jax.experimental import pallas as pl
from jax.experimental.pallas import tpu as pltpu
from jax.experimental.pallas import tpu_sc as plsc

N_EMB = 1000000
EMB_DIM = 32
EMB_SCALE = 10.0

B_TOTAL = 16384 * 50
NUM_WORKERS = 32
B_PER_W = B_TOTAL // NUM_WORKERS
CHUNK = 1280
NCHUNK = B_PER_W // CHUNK


def _sc_kernel(x_hbm, table_hbm, out_hbm, idx_v, rows_v, sem):
    cid = lax.axis_index("c")
    sid = lax.axis_index("s")
    wid = sid * 2 + cid
    base = wid * B_PER_W

    @pl.loop(0, NCHUNK)
    def _chunk(g):
        off = base + g * CHUNK
        pltpu.sync_copy(x_hbm.at[pl.ds(off, CHUNK)], idx_v)
        pltpu.async_copy(table_hbm.at[idx_v], rows_v, sem).wait()

        @pl.loop(0, CHUNK, unroll=4)
        def _scale(i):
            rows_v[i, pl.ds(0, 16)] = rows_v[i, pl.ds(0, 16)] * EMB_SCALE
            rows_v[i, pl.ds(16, 16)] = rows_v[i, pl.ds(16, 16)] * EMB_SCALE

        pltpu.sync_copy(rows_v, out_hbm.at[pl.ds(off, CHUNK)])


@jax.jit
def _scaled_embedding(x_flat, table):
    mesh = plsc.VectorSubcoreMesh(core_axis_name="c", subcore_axis_name="s")
    return pl.kernel(
        _sc_kernel,
        out_type=jax.ShapeDtypeStruct((B_TOTAL, EMB_DIM), jnp.float32),
        mesh=mesh,
        scratch_types=[
            pltpu.VMEM((CHUNK,), jnp.int32),
            pltpu.VMEM((CHUNK, EMB_DIM), jnp.float32),
            pltpu.SemaphoreType.DMA,
        ],
        compiler_params=pltpu.CompilerParams(use_tc_tiling_on_sc=False),
    )(x_flat, table)


def kernel(x, table):
    out = _scaled_embedding(x.reshape(-1).astype(jnp.int32), table)
    return out.reshape(x.shape + (EMB_DIM,))


# speedup vs baseline: 1.2544x; 1.2297x over previous
import jax
import jax.numpy as jnp
from jax import lax
from jax.experimental import pallas as pl
from jax.experimental.pallas import tpu as pltpu
from jax.experimental.pallas import tpu_sc as plsc

N_EMB = 1000000
EMB_DIM = 32
EMB_SCALE = 10.0

N_B = 16384
N_S = 50
LANES = 16
BT = 128
N_BT = N_B // BT
NUM_WORKERS = 32
BT_PER_W = N_BT // NUM_WORKERS
XBLK = BT * N_S


def _sc_kernel(x_hbm, table_hbm, out_hbm, x_blk, idx_buf, rows_v, out_t, sem):
    cid = lax.axis_index("c")
    sid = lax.axis_index("s")
    wid = sid * 2 + cid

    lane = lax.iota(jnp.int32, LANES)
    lane50 = lane * N_S
    lane32 = lane * EMB_DIM

    @pl.loop(0, BT_PER_W)
    def _bt(j):
        tc = wid * BT_PER_W + j
        pltpu.sync_copy(x_hbm.at[pl.ds(tc * XBLK, XBLK)], x_blk)

        @pl.loop(0, N_S)
        def _s(s):
            for g in range(BT // LANES):
                xv = plsc.load_gather(x_blk, [lane50 + (g * LANES * N_S + s)])
                idx_buf[pl.ds(g * LANES, LANES)] = xv
            pltpu.async_copy(table_hbm.at[idx_buf], rows_v, sem).wait()
            for d in range(EMB_DIM):
                tr, su = d // 8, d % 8
                for g in range(BT // LANES):
                    v = plsc.load_gather(rows_v, [lane + g * LANES,
                                                  jnp.full((LANES,), d, jnp.int32)])
                    out_t[tr, su, pl.ds(g * LANES, LANES)] = v * EMB_SCALE
            pltpu.sync_copy(out_t, out_hbm.at[s, :, tc])


@jax.jit
def _scaled_embedding(x2d, table):
    mesh = plsc.VectorSubcoreMesh(core_axis_name="c", subcore_axis_name="s")
    out5 = pl.kernel(
        _sc_kernel,
        out_type=jax.ShapeDtypeStruct((N_S, 4, N_BT, 8, BT), jnp.float32),
        mesh=mesh,
        scratch_types=[
            pltpu.VMEM((XBLK,), jnp.int32),
            pltpu.VMEM((BT,), jnp.int32),
            pltpu.VMEM((BT, EMB_DIM), jnp.float32),
            pltpu.VMEM((4, 8, BT), jnp.float32),
            pltpu.SemaphoreType.DMA,
        ],
        compiler_params=pltpu.CompilerParams(
            use_tc_tiling_on_sc=False, needs_layout_passes=False
        ),
    )(x2d.reshape(-1), table)
    return out5.transpose((2, 4, 0, 1, 3)).reshape(N_B, N_S, EMB_DIM)


def kernel(x, table):
    return _scaled_embedding(x, table)


# speedup vs baseline: 1.4151x; 1.1281x over previous
import jax
import jax.numpy as jnp
from jax import lax
from jax.experimental import pallas as pl
from jax.experimental.pallas import tpu as pltpu
from jax.experimental.pallas import tpu_sc as plsc

N_EMB = 1000000
EMB_DIM = 32
EMB_SCALE = 10.0

N_B = 16384
N_S = 50
LANES = 16
BT = 128
N_BT = N_B // BT
NUM_WORKERS = 32
BT_PER_W = N_BT // NUM_WORKERS
XBLK = BT * N_S
NGRP = BT // LANES


def _sc_kernel(x_hbm, table_hbm, out_hbm,
               x_blk, idx_a, idx_b, rows_a, rows_b, out_a, out_b,
               sem_ga, sem_gb, sem_oa, sem_ob):
    cid = lax.axis_index("c")
    sid = lax.axis_index("s")
    wid = sid * 2 + cid

    lane = lax.iota(jnp.int32, LANES)
    lane50 = lane * N_S

    def extract_and_fire(s, idx_v, rows_v, sem):
        for g in range(NGRP):
            xv = plsc.load_gather(x_blk, [lane50 + (g * LANES * N_S + s)])
            idx_v[pl.ds(g * LANES, LANES)] = xv
        return pltpu.async_copy(table_hbm.at[idx_v], rows_v, sem)

    def transpose_and_write(s, tc, idx_v, rows_v, sem_g, out_t, sem_o, drain_out):
        pltpu.make_async_copy(table_hbm.at[idx_v], rows_v, sem_g).wait()
        if drain_out is not None:
            drain_out()
        for g in range(NGRP):
            ridx = lane + g * LANES
            for d in range(EMB_DIM):
                v = plsc.load_gather(
                    rows_v, [ridx, jnp.full((LANES,), d, jnp.int32)])
                out_t[d // 8, d % 8, pl.ds(g * LANES, LANES)] = v * EMB_SCALE
        return pltpu.async_copy(out_t, out_hbm.at[s, :, tc], sem_o)

    @pl.loop(0, BT_PER_W)
    def _bt(j):
        tc = wid * BT_PER_W + j
        pltpu.sync_copy(x_hbm.at[pl.ds(tc * XBLK, XBLK)], x_blk)

        extract_and_fire(0, idx_a, rows_a, sem_ga)

        def drain(out_t, sem_o, cond):
            def _do():
                @pl.when(cond)
                def _w():
                    pltpu.make_async_copy(
                        out_t, out_hbm.at[0, :, tc], sem_o).wait()
            return _do

        @pl.loop(0, N_S // 2)
        def _p(p):
            s0 = 2 * p
            extract_and_fire(s0 + 1, idx_b, rows_b, sem_gb)
            transpose_and_write(s0, tc, idx_a, rows_a, sem_ga, out_a, sem_oa,
                                drain(out_a, sem_oa, s0 >= 2))

            @pl.when(s0 + 2 < N_S)
            def _nx():
                extract_and_fire(s0 + 2, idx_a, rows_a, sem_ga)

            transpose_and_write(s0 + 1, tc, idx_b, rows_b, sem_gb, out_b, sem_ob,
                                drain(out_b, sem_ob, s0 >= 1))

        pltpu.make_async_copy(out_a, out_hbm.at[0, :, tc], sem_oa).wait()
        pltpu.make_async_copy(out_b, out_hbm.at[0, :, tc], sem_ob).wait()


@jax.jit
def _scaled_embedding(x2d, table):
    mesh = plsc.VectorSubcoreMesh(core_axis_name="c", subcore_axis_name="s")
    out5 = pl.kernel(
        _sc_kernel,
        out_type=jax.ShapeDtypeStruct((N_S, 4, N_BT, 8, BT), jnp.float32),
        mesh=mesh,
        scratch_types=[
            pltpu.VMEM((XBLK,), jnp.int32),
            pltpu.VMEM((BT,), jnp.int32),
            pltpu.VMEM((BT,), jnp.int32),
            pltpu.VMEM((BT, EMB_DIM), jnp.float32),
            pltpu.VMEM((BT, EMB_DIM), jnp.float32),
            pltpu.VMEM((4, 8, BT), jnp.float32),
            pltpu.VMEM((4, 8, BT), jnp.float32),
            pltpu.SemaphoreType.DMA,
            pltpu.SemaphoreType.DMA,
            pltpu.SemaphoreType.DMA,
            pltpu.SemaphoreType.DMA,
        ],
        compiler_params=pltpu.CompilerParams(
            use_tc_tiling_on_sc=False, needs_layout_passes=False
        ),
    )(x2d.reshape(-1), table)
    return out5.transpose((2, 4, 0, 1, 3)).reshape(N_B, N_S, EMB_DIM)


def kernel(x, table):
    return _scaled_embedding(x, table)
